# Initial kernel scaffold; baseline (speedup 1.0000x reference)
import jax
import jax.numpy as jnp
from jax.experimental import pallas as pl


def kernel(x, table):
    raise NotImplementedError("write your pallas kernel here")


# speedup vs baseline: 1.1089x; 1.1089x over previous
import functools

import jax
import jax.numpy as jnp
from jax import lax
from jax.experimental import pallas as pl
from jax.experimental.pallas import tpu as pltpu
from jax.experimental.pallas import tpu_sc as plsc

_C = 128
_NBUF = 4


@functools.partial(jax.jit, static_argnums=(2, 3, 4))
def _gather_rows(idx, table, nw, nchunks, d):
    n = nw * nchunks * _C
    b_per_w = nchunks * _C
    mesh = plsc.VectorSubcoreMesh(core_axis_name="c", subcore_axis_name="s")

    @functools.partial(
        pl.kernel,
        out_type=jax.ShapeDtypeStruct((n, d), jnp.float32),
        mesh=mesh,
        scratch_types=[
            pltpu.VMEM((nchunks, _C), jnp.int32),
            pltpu.VMEM((_NBUF, _C, d), jnp.float32),
            [pltpu.SemaphoreType.DMA] * _NBUF,
            [pltpu.SemaphoreType.DMA] * _NBUF,
        ],
        compiler_params=pltpu.CompilerParams(use_tc_tiling_on_sc=False),
    )
    def k(idx_hbm, table_hbm, out_hbm, idx_v, rows_v, gsems, osems):
        wid = lax.axis_index("s") * 2 + lax.axis_index("c")
        base = wid * b_per_w

        pltpu.sync_copy(idx_hbm.at[wid], idx_v)

        def fire_gather(jj, s):
            pltpu.async_copy(table_hbm.at[idx_v.at[jj]], rows_v.at[s], gsems[s])

        def wait_gather(jj, s):
            pltpu.make_async_copy(
                table_hbm.at[idx_v.at[jj]], rows_v.at[s], gsems[s]).wait()

        def fire_write(jj, s):
            pltpu.async_copy(
                rows_v.at[s], out_hbm.at[pl.ds(base + jj * _C, _C)], osems[s])

        def wait_write(jj, s):
            pltpu.make_async_copy(
                rows_v.at[s], out_hbm.at[pl.ds(base + jj * _C, _C)],
                osems[s]).wait()

        for s in range(_NBUF):
            fire_gather(s, s)

        @pl.loop(0, nchunks, step=_NBUF)
        def _(j):
            for s in range(_NBUF):
                jj = j + s
                wait_gather(jj, s)
                fire_write(jj, s)

                @pl.when(jj + _NBUF < nchunks)
                def _():
                    wait_write(jj, s)
                    fire_gather(jj + _NBUF, s)

        for i in range(_NBUF):
            jj = nchunks - _NBUF + i
            wait_write(jj, jj % _NBUF)

    return k(idx, table)


def kernel(x, table):
    b, h = x.shape
    v, d = table.shape
    n = b * h
    nw = 32
    chunk = nw * _C * _NBUF
    n_pad = ((n + chunk - 1) // chunk) * chunk
    idx = x.reshape(n).astype(jnp.int32)
    if n_pad != n:
        idx = jnp.concatenate([idx, jnp.zeros(n_pad - n, jnp.int32)])
    nchunks = n_pad // (nw * _C)
    idx = idx.reshape(nw, nchunks, _C)
    out = _gather_rows(idx, table, nw, nchunks, d)
    return out[:n].reshape(b, h, d)
